# baseline (speedup 1.0000x reference)
import functools

import jax
import jax.numpy as jnp
from jax import lax
from jax.experimental import pallas as pl
from jax.experimental.pallas import tpu as pltpu
from jax.experimental.pallas import tpu_sc as plsc

BATCH = 4096
SEQLEN = 200
NUM_HID = 64
NC = 2
NS = 16
NW = NC * NS
BB = BATCH // 128
LANES = 16
HID_VECS = NUM_HID // LANES
SI = SEQLEN // 8


def _make_kernel():
    mesh = plsc.VectorSubcoreMesh(core_axis_name="c", subcore_axis_name="s")

    @functools.partial(
        pl.kernel,
        out_type=jax.ShapeDtypeStruct((SEQLEN, 8, BB, 8, 128), jnp.float32),
        mesh=mesh,
        scratch_types=[
            pltpu.VMEM((SI, 8, 128), jnp.int32),
            pltpu.VMEM((SEQLEN, NUM_HID), jnp.float32),
            pltpu.VMEM((128, NUM_HID), jnp.float32),
            pltpu.VMEM((128, NUM_HID), jnp.float32),
            pltpu.VMEM((8, 8, 129), jnp.float32),
            pltpu.VMEM((8, 8, 129), jnp.float32),
            pltpu.SemaphoreType.DMA,
            pltpu.SemaphoreType.DMA,
            pltpu.SemaphoreType.DMA,
        ],
        compiler_params=pltpu.CompilerParams(
            use_tc_tiling_on_sc=False, needs_layout_passes=False),
    )
    def k(x4_hbm, emb_hbm, pos_hbm, out_hbm, idx_all, pos_v, tok_a, tok_b,
          st_a, st_b, gsem, osem_a, osem_b):
        w = lax.axis_index("s") * NC + lax.axis_index("c")
        pltpu.sync_copy(x4_hbm.at[:, w], idx_all)
        pltpu.sync_copy(pos_hbm, pos_v)

        iota = lax.iota(jnp.int32, 16)
        hh_k = [iota // 8 + 2 * kk for kk in range(HID_VECS)]
        hl_base = iota % 8
        zeros = iota * 0

        def start_gather(s, tok):
            return pltpu.async_copy(
                emb_hbm.at[idx_all.at[s // 8, s % 8]], tok, gsem)

        def wait_gather(tok):
            pltpu.make_async_copy(
                emb_hbm.at[idx_all.at[0, 0]], tok, gsem).wait()

        def fire_outs(s, st, osem):
            pltpu.async_copy(st.at[:, :, pl.ds(0, 128)],
                             out_hbm.at[s, :, w], osem)

        def wait_outs(st, osem):
            pltpu.make_async_copy(st.at[:, :, pl.ds(0, 128)],
                                  out_hbm.at[0, :, w], osem).wait()

        def compute(s, tok, st):
            posr = [pos_v[s, pl.ds(kk * LANES, LANES)]
                    for kk in range(HID_VECS)]

            @plsc.parallel_loop(0, 128, step=1, unroll=16)
            def tok_body(t):
                col = zeros + t
                for kk in range(HID_VECS):
                    v = tok[t, pl.ds(kk * LANES, LANES)] + posr[kk]
                    plsc.store_scatter(st, [hh_k[kk], hl_base, col], v)

        start_gather(0, tok_a)

        def body(j2, carry):
            s0 = 2 * j2
            s1 = s0 + 1
            wait_gather(tok_a)
            start_gather(s1, tok_b)

            @pl.when(j2 > 0)
            def _():
                wait_outs(st_a, osem_a)

            compute(s0, tok_a, st_a)
            fire_outs(s0, st_a, osem_a)

            wait_gather(tok_b)

            @pl.when(j2 < SEQLEN // 2 - 1)
            def _():
                start_gather(s0 + 2, tok_a)

            @pl.when(j2 > 0)
            def _():
                wait_outs(st_b, osem_b)

            compute(s1, tok_b, st_b)
            fire_outs(s1, st_b, osem_b)
            return carry

        lax.fori_loop(0, SEQLEN // 2, body, 0)
        wait_outs(st_a, osem_a)
        wait_outs(st_b, osem_b)

    return k


_kernel_call = _make_kernel()


def kernel(x, emb_table, pos_table):
    x4 = x.T.reshape(SI, 8, BB, 128).transpose(0, 2, 1, 3)
    out5 = _kernel_call(x4, emb_table, pos_table)
    return (out5.transpose(2, 4, 0, 1, 3)
            .reshape(BATCH, SEQLEN, NUM_HID))


# speedup vs baseline: 1.0025x; 1.0025x over previous
import functools

import jax
import jax.numpy as jnp
from jax import lax
from jax.experimental import pallas as pl
from jax.experimental.pallas import tpu as pltpu
from jax.experimental.pallas import tpu_sc as plsc

BATCH = 4096
SEQLEN = 200
NUM_HID = 64
NC = 2
NS = 16
NW = NC * NS
BB = BATCH // 128
LANES = 16
HID_VECS = NUM_HID // LANES
SI = SEQLEN // 8


def _make_kernel():
    mesh = plsc.VectorSubcoreMesh(core_axis_name="c", subcore_axis_name="s")

    @functools.partial(
        pl.kernel,
        out_type=jax.ShapeDtypeStruct((SEQLEN, 8, BB, 8, 128), jnp.float32),
        mesh=mesh,
        scratch_types=[
            pltpu.VMEM((SI, 8, 128), jnp.int32),
            pltpu.VMEM((SEQLEN, NUM_HID), jnp.float32),
            pltpu.VMEM((128, NUM_HID), jnp.float32),
            pltpu.VMEM((128, NUM_HID), jnp.float32),
            pltpu.VMEM((8, 8, 129), jnp.float32),
            pltpu.VMEM((8, 8, 129), jnp.float32),
            pltpu.SemaphoreType.DMA,
            pltpu.SemaphoreType.DMA,
            pltpu.SemaphoreType.DMA,
        ],
        compiler_params=pltpu.CompilerParams(
            use_tc_tiling_on_sc=False, needs_layout_passes=False),
    )
    def k(x4_hbm, emb_hbm, pos_hbm, out_hbm, idx_all, pos_v, tok_a, tok_b,
          st_a, st_b, gsem, osem_a, osem_b):
        w = lax.axis_index("s") * NC + lax.axis_index("c")
        pltpu.sync_copy(x4_hbm.at[:, w], idx_all)
        pltpu.sync_copy(pos_hbm, pos_v)

        iota = lax.iota(jnp.int32, 16)
        hh_k = [iota // 8 + 2 * kk for kk in range(HID_VECS)]
        hl_base = iota % 8
        zeros = iota * 0

        def start_gather(s, tok):
            return pltpu.async_copy(
                emb_hbm.at[idx_all.at[s // 8, s % 8]], tok, gsem)

        def wait_gather(tok):
            pltpu.make_async_copy(
                emb_hbm.at[idx_all.at[0, 0]], tok, gsem).wait()

        def fire_outs(s, st, osem):
            pltpu.async_copy(st.at[:, :, pl.ds(0, 128)],
                             out_hbm.at[s, :, w], osem)

        def wait_outs(st, osem):
            pltpu.make_async_copy(st.at[:, :, pl.ds(0, 128)],
                                  out_hbm.at[0, :, w], osem).wait()

        def compute(s, tok, st):
            posr = [pos_v[s, pl.ds(kk * LANES, LANES)]
                    for kk in range(HID_VECS)]

            @plsc.parallel_loop(0, 128, step=1, unroll=8)
            def tok_body(t):
                col = zeros + t
                for kk in range(HID_VECS):
                    v = tok[t, pl.ds(kk * LANES, LANES)] + posr[kk]
                    plsc.store_scatter(st, [hh_k[kk], hl_base, col], v)

        start_gather(0, tok_a)

        def body(j2, carry):
            s0 = 2 * j2
            s1 = s0 + 1
            wait_gather(tok_a)
            start_gather(s1, tok_b)

            @pl.when(j2 > 0)
            def _():
                wait_outs(st_a, osem_a)

            compute(s0, tok_a, st_a)
            fire_outs(s0, st_a, osem_a)

            wait_gather(tok_b)

            @pl.when(j2 < SEQLEN // 2 - 1)
            def _():
                start_gather(s0 + 2, tok_a)

            @pl.when(j2 > 0)
            def _():
                wait_outs(st_b, osem_b)

            compute(s1, tok_b, st_b)
            fire_outs(s1, st_b, osem_b)
            return carry

        lax.fori_loop(0, SEQLEN // 2, body, 0)
        wait_outs(st_a, osem_a)
        wait_outs(st_b, osem_b)

    return k


_kernel_call = _make_kernel()


def kernel(x, emb_table, pos_table):
    x4 = x.T.reshape(SI, 8, BB, 128).transpose(0, 2, 1, 3)
    out5 = _kernel_call(x4, emb_table, pos_table)
    return (out5.transpose(2, 4, 0, 1, 3)
            .reshape(BATCH, SEQLEN, NUM_HID))
